# baseline (speedup 1.0000x reference)
import functools

import jax
import jax.numpy as jnp
import numpy as np
from jax.experimental import pallas as pl

_INTERPRET = False


def _pc(body, **kw):
    return pl.pallas_call(body, interpret=_INTERPRET, **kw)


def _fps_body(x_ref, y_ref, z_ref, idx_ref, px_ref, py_ref, pz_ref, *, M):
    x = x_ref[...]
    y = y_ref[...]
    z = z_ref[...]
    b, n = x.shape
    iota = jax.lax.broadcasted_iota(jnp.int32, (b, n), 1)
    miota = jax.lax.broadcasted_iota(jnp.int32, (b, M), 1)
    x0 = x[:, 0:1]
    y0 = y[:, 0:1]
    z0 = z[:, 0:1]
    idxs0 = jnp.zeros((b, M), jnp.int32)
    pxs0 = jnp.where(miota == 0, x0, 0.0)
    pys0 = jnp.where(miota == 0, y0, 0.0)
    pzs0 = jnp.where(miota == 0, z0, 0.0)
    dists0 = jnp.full((b, n), 1e10, jnp.float32)

    def body(i, st):
        dists, lx, ly, lz, idxs, pxs, pys, pzs = st
        dx = x - lx
        dy = y - ly
        dz = z - lz
        d = dx * dx + dy * dy + dz * dz
        dists = jnp.minimum(dists, d)
        m = jnp.max(dists, axis=1, keepdims=True)
        amax = jnp.min(jnp.where(dists == m, iota, n), axis=1, keepdims=True)
        sel = iota == amax
        nlx = jnp.sum(jnp.where(sel, x, 0.0), axis=1, keepdims=True)
        nly = jnp.sum(jnp.where(sel, y, 0.0), axis=1, keepdims=True)
        nlz = jnp.sum(jnp.where(sel, z, 0.0), axis=1, keepdims=True)
        wr = miota == i
        idxs = jnp.where(wr, amax, idxs)
        pxs = jnp.where(wr, nlx, pxs)
        pys = jnp.where(wr, nly, pys)
        pzs = jnp.where(wr, nlz, pzs)
        return (dists, nlx, nly, nlz, idxs, pxs, pys, pzs)

    st = (dists0, x0, y0, z0, idxs0, pxs0, pys0, pzs0)
    st = jax.lax.fori_loop(1, M, body, st)
    idx_ref[...] = st[4]
    px_ref[...] = st[5]
    py_ref[...] = st[6]
    pz_ref[...] = st[7]


def _fps(x, y, z, M):
    b, _ = x.shape
    outs = _pc(
        functools.partial(_fps_body, M=M),
        out_shape=[
            jax.ShapeDtypeStruct((b, M), jnp.int32),
            jax.ShapeDtypeStruct((b, M), jnp.float32),
            jax.ShapeDtypeStruct((b, M), jnp.float32),
            jax.ShapeDtypeStruct((b, M), jnp.float32),
        ],
    )(x, y, z)
    return outs


def _bq_body(qx_ref, qy_ref, qz_ref, x_ref, y_ref, z_ref, idx_ref, *, K, r2):
    qx = qx_ref[0]
    qy = qy_ref[0]
    qz = qz_ref[0]
    x = x_ref[0]
    y = y_ref[0]
    z = z_ref[0]
    qb = qx.shape[0]
    n = x.shape[1]
    dx = qx - x
    dy = qy - y
    dz = qz - z
    d2 = dx * dx + dy * dy + dz * dz
    mask = d2 < r2
    rank = mask.astype(jnp.int32)
    sh = 1
    while sh < n:
        shifted = jnp.concatenate(
            [jnp.zeros((qb, sh), jnp.int32), rank[:, :n - sh]], axis=1)
        rank = rank + shifted
        sh *= 2
    iota_n = jax.lax.broadcasted_iota(jnp.int32, (qb, n), 1)
    kcol = jax.lax.broadcasted_iota(jnp.int32, (qb, K), 1)
    rkm = jnp.where(mask, rank, 0)

    def body(k, accI):
        sel = rkm == k + 1
        idxk = jnp.sum(jnp.where(sel, iota_n, 0), axis=1, keepdims=True)
        return jnp.where(kcol == k, idxk, accI)

    accI = jax.lax.fori_loop(0, K, body, jnp.zeros((qb, K), jnp.int32))
    idx_ref[0] = accI


def _bq(qx, qy, qz, x, y, z, K, radius, Qb):
    b, Q = qx.shape
    n = x.shape[1]
    r2 = float(np.float32(radius) * np.float32(radius))
    q3 = lambda a: a[..., None]
    p3 = lambda a: a[:, None, :]
    qspec = pl.BlockSpec((1, Qb, 1), lambda i, j: (i, j, 0))
    pspec = pl.BlockSpec((1, 1, n), lambda i, j: (i, 0, 0))
    ospec = pl.BlockSpec((1, Qb, K), lambda i, j: (i, j, 0))
    return _pc(
        functools.partial(_bq_body, K=K, r2=r2),
        grid=(b, Q // Qb),
        in_specs=[qspec, qspec, qspec, pspec, pspec, pspec],
        out_specs=ospec,
        out_shape=jax.ShapeDtypeStruct((b, Q, K), jnp.int32),
    )(q3(qx), q3(qy), q3(qz), p3(x), p3(y), p3(z))


def _acc_init(sum_ref):
    @pl.when(jnp.logical_and(pl.program_id(0) == 0, pl.program_id(1) == 0))
    def _():
        sum_ref[...] = jnp.zeros_like(sum_ref)


def _acc_update(y, sum_ref):
    sum_ref[...] += jnp.sum(y, axis=1, keepdims=True)


def _l1_body(w_ref, b_ref, src_ref, idx_ref, qx_ref, qy_ref, qz_ref,
             y_ref, sum_ref, *, nc):
    _acc_init(sum_ref)
    src = src_ref[0]
    cin, n = src.shape
    idxb = idx_ref[0]
    t = idxb.shape[1]
    L = n // nc
    g = None
    for c in range(nc):
        rowi = jax.lax.broadcasted_iota(jnp.int32, (L, t), 0) + c * L
        oh = (rowi == idxb).astype(jnp.float32)
        part = jnp.dot(src[:, c * L:(c + 1) * L], oh,
                       preferred_element_type=jnp.float32,
                       precision=jax.lax.Precision.HIGHEST)
        g = part if g is None else g + part
    parts = [g[0:1] - qx_ref[0], g[1:2] - qy_ref[0], g[2:3] - qz_ref[0]]
    if cin > 3:
        parts.append(g[3:])
    xcat = jnp.concatenate(parts, axis=0)
    y = (jnp.dot(w_ref[...], xcat, preferred_element_type=jnp.float32)
         + b_ref[...])
    y_ref[0] = y
    _acc_update(y, sum_ref)


def _layer_body(scale_ref, shift_ref, w_ref, b_ref, x_ref, y_ref, sum_ref):
    _acc_init(sum_ref)
    x = x_ref[0]
    xn = jnp.maximum(x * scale_ref[...] + shift_ref[...], 0.0)
    y = jnp.dot(w_ref[...], xn, preferred_element_type=jnp.float32) + b_ref[...]
    y_ref[0] = y
    _acc_update(y, sum_ref)


def _var_body(mean_ref, y_ref, ssq_ref):
    _acc_init(ssq_ref)
    yc = y_ref[0] - mean_ref[...]
    ssq_ref[...] += jnp.sum(yc * yc, axis=1, keepdims=True)


def _var(mean, y, T):
    b, O, KS = y.shape
    return _pc(
        _var_body,
        grid=(b, KS // T),
        in_specs=[pl.BlockSpec((O, 1), lambda i, j: (0, 0)),
                  pl.BlockSpec((1, O, T), lambda i, j: (i, 0, j))],
        out_specs=pl.BlockSpec((O, 1), lambda i, j: (0, 0)),
        out_shape=jax.ShapeDtypeStruct((O, 1), jnp.float32),
    )(mean, y)


def _stat_specs(O):
    return ([pl.BlockSpec((O, 1), lambda *a: (0, 0))],
            [jax.ShapeDtypeStruct((O, 1), jnp.float32)])


def _run_l1(W, bb, src, idxf, qxf, qyf, qzf, T, nc):
    b, KS = idxf.shape
    O, Cin = W.shape
    cs, n = src.shape[1], src.shape[2]
    g3 = lambda a: a[:, None, :]
    gspec = pl.BlockSpec((1, 1, T), lambda i, j: (i, 0, j))
    sspec, sshape = _stat_specs(O)
    return _pc(
        functools.partial(_l1_body, nc=nc),
        grid=(b, KS // T),
        in_specs=[pl.BlockSpec((O, Cin), lambda i, j: (0, 0)),
                  pl.BlockSpec((O, 1), lambda i, j: (0, 0)),
                  pl.BlockSpec((1, cs, n), lambda i, j: (i, 0, 0)),
                  gspec, gspec, gspec, gspec],
        out_specs=[pl.BlockSpec((1, O, T), lambda i, j: (i, 0, j))] + sspec,
        out_shape=[jax.ShapeDtypeStruct((b, O, KS), jnp.float32)] + sshape,
    )(W, bb[:, None], src, g3(idxf), g3(qxf), g3(qyf), g3(qzf))


def _run_layer(scale, shift, W, bb, x, T):
    b, Cin, KS = x.shape
    O = W.shape[0]
    sspec, sshape = _stat_specs(O)
    return _pc(
        _layer_body,
        grid=(b, KS // T),
        in_specs=[pl.BlockSpec((Cin, 1), lambda i, j: (0, 0)),
                  pl.BlockSpec((Cin, 1), lambda i, j: (0, 0)),
                  pl.BlockSpec((O, Cin), lambda i, j: (0, 0)),
                  pl.BlockSpec((O, 1), lambda i, j: (0, 0)),
                  pl.BlockSpec((1, Cin, T), lambda i, j: (i, 0, j))],
        out_specs=[pl.BlockSpec((1, O, T), lambda i, j: (i, 0, j))] + sspec,
        out_shape=[jax.ShapeDtypeStruct((b, O, KS), jnp.float32)] + sshape,
    )(scale, shift, W, bb[:, None], x)


def _affine(sums, y, count, layer, T):
    mean = sums / count
    var = _var(mean, y, T) / count
    inv = 1.0 / jnp.sqrt(var + 1e-5)
    scale = layer['gamma'][:, None] * inv
    shift = layer['beta'][:, None] - mean * scale
    return scale, shift


def _pool_ks_body(scale_ref, shift_ref, y_ref, f_ref, *, K, S):
    y = y_ref[0]
    m = y[:, 0:S]
    for k in range(1, K):
        m = jnp.maximum(m, y[:, k * S:(k + 1) * S])
    f_ref[0] = jnp.maximum(m * scale_ref[...] + shift_ref[...], 0.0)


def _pool_sk_body(scale_ref, shift_ref, y_ref, f_ref, *, K, S):
    y = y_ref[0]
    o = y.shape[0]
    m = jnp.max(y.reshape(o, S, K), axis=2)
    f_ref[0] = jnp.maximum(m * scale_ref[...] + shift_ref[...], 0.0)


def _pool(scale, shift, y, K, S, neighbor_minor):
    b, O, KS = y.shape
    body = _pool_sk_body if neighbor_minor else _pool_ks_body
    return _pc(
        functools.partial(body, K=K, S=S),
        grid=(b,),
        in_specs=[pl.BlockSpec((O, 1), lambda i: (0, 0)),
                  pl.BlockSpec((O, 1), lambda i: (0, 0)),
                  pl.BlockSpec((1, O, KS), lambda i: (i, 0, 0))],
        out_specs=pl.BlockSpec((1, O, S), lambda i: (i, 0, 0)),
        out_shape=jax.ShapeDtypeStruct((b, O, S), jnp.float32),
    )(scale, shift, y)


def _mlp(layers, count, y1, s1, T, pool_args):
    sc, sh = _affine(s1, y1, count, layers[0], T)
    y2, s2 = _run_layer(sc, sh, layers[1]['W'], layers[1]['b'], y1, T)
    sc, sh = _affine(s2, y2, count, layers[1], T)
    y3, s3 = _run_layer(sc, sh, layers[2]['W'], layers[2]['b'], y2, T)
    sc, sh = _affine(s3, y3, count, layers[2], T)
    K, S, neighbor_minor = pool_args
    return _pool(sc, sh, y3, K, S, neighbor_minor)


def _stage(x, y, z, feat, layers, S, K, radius, Qb, neighbor_minor, nc, Tmlp):
    b = x.shape[0]
    _, px, py, pz = _fps(x, y, z, S)
    idx = _bq(px, py, pz, x, y, z, K=K, radius=radius, Qb=Qb)
    if neighbor_minor:
        fl = lambda a: a.reshape(b, S * K)
        tile = lambda a: jnp.broadcast_to(a[:, :, None], (b, S, K)).reshape(
            b, S * K)
    else:
        fl = lambda a: a.transpose(0, 2, 1).reshape(b, K * S)
        tile = lambda a: jnp.broadcast_to(a[:, None, :], (b, K, S)).reshape(
            b, K * S)
    src = jnp.stack([x, y, z], axis=1)
    if feat is not None:
        src = jnp.concatenate([src, feat], axis=1)
    y1, s1 = _run_l1(layers[0]['W'], layers[0]['b'], src, fl(idx),
                     tile(px), tile(py), tile(pz), T=4096, nc=nc)
    count = np.float32(b * K * S)
    f = _mlp(layers, count, y1, s1, Tmlp, (K, S, neighbor_minor))
    return (px, py, pz), f


@jax.jit
def kernel(xyz, params):
    x = xyz[:, :, 0]
    y = xyz[:, :, 1]
    z = xyz[:, :, 2]
    (px1, py1, pz1), f1 = _stage(x, y, z, None, params['sa1'],
                                 S=512, K=32, radius=0.1, Qb=128,
                                 neighbor_minor=False, nc=8, Tmlp=4096)
    (px2, py2, pz2), f2 = _stage(px1, py1, pz1, f1, params['sa2'],
                                 S=128, K=64, radius=0.25, Qb=128,
                                 neighbor_minor=False, nc=1, Tmlp=4096)
    (px3, py3, pz3), f3 = _stage(px2, py2, pz2, f2, params['sa3'],
                                 S=32, K=128, radius=0.5, Qb=32,
                                 neighbor_minor=True, nc=1, Tmlp=2048)
    xyz1 = jnp.stack([px1, py1, pz1], axis=-1)
    xyz2 = jnp.stack([px2, py2, pz2], axis=-1)
    xyz3 = jnp.stack([px3, py3, pz3], axis=-1)
    return (xyz1, f1, xyz2, f2, xyz3, f3)
